# Initial kernel scaffold; baseline (speedup 1.0000x reference)
import jax
import jax.numpy as jnp
from jax.experimental import pallas as pl


def kernel(x, edge_index, W, b):
    raise NotImplementedError("write your pallas kernel here")


# speedup vs baseline: 43.1158x; 43.1158x over previous
import functools

import jax
import jax.numpy as jnp
from jax import lax
from jax.experimental import pallas as pl
from jax.experimental.pallas import tpu as pltpu
from jax.experimental.pallas import tpu_sc as plsc

N = 10000
E = 320000
D = 128
NC = 2
NS = 16
NW = NC * NS
EPT = E // NW
B = 125
NBT = EPT // B
ACC_N = 10240
ROWS_PT = ACC_N // NS

_mesh = plsc.VectorSubcoreMesh(core_axis_name="c", subcore_axis_name="s")


@functools.partial(
    pl.kernel,
    out_type=jax.ShapeDtypeStruct((NC, ACC_N), jnp.float32),
    mesh=_mesh,
    scratch_types=[
        pltpu.VMEM((NBT, B), jnp.int32),
        pltpu.VMEM((B,), jnp.float32),
        pltpu.VMEM_SHARED((ACC_N,), jnp.float32),
        pltpu.SemaphoreType.DMA,
    ],
)
def _deg_call(dst2_hbm, ones_hbm, z_hbm, out_hbm, dstv, onesb, dacc, sem):
    c = lax.axis_index("c")
    s = lax.axis_index("s")
    wid = c * NS + s
    pltpu.sync_copy(dst2_hbm.at[pl.ds(wid * NBT, NBT)], dstv)
    pltpu.sync_copy(ones_hbm, onesb)
    pltpu.sync_copy(z_hbm, dacc.at[pl.ds(s * ROWS_PT, ROWS_PT)])
    plsc.subcore_barrier()

    def fire(j, carry):
        pltpu.async_copy(onesb, dacc.at[dstv.at[j]], sem, add=True)
        return carry

    lax.fori_loop(0, NBT, fire, 0)

    def drain(j, carry):
        pltpu.make_async_copy(onesb, dacc.at[dstv.at[j]], sem).wait()
        return carry

    lax.fori_loop(0, NBT, drain, 0)
    plsc.subcore_barrier()
    pltpu.sync_copy(dacc.at[pl.ds(s * ROWS_PT, ROWS_PT)],
                    out_hbm.at[c, pl.ds(s * ROWS_PT, ROWS_PT)])


_R = 1024


def _hs_body(x_ref, w_ref, degp_ref, hs_ref):
    i = pl.program_id(0)
    deg = (degp_ref[0, pl.ds(i * _R, _R)] + degp_ref[1, pl.ds(i * _R, _R)]
           + 1.0)
    dis = lax.rsqrt(deg)
    h = jnp.dot(x_ref[...], w_ref[...], preferred_element_type=jnp.float32)
    hs_ref[...] = h * dis[:, None]


def _hs_call(x, W, degp):
    return pl.pallas_call(
        _hs_body,
        grid=(ACC_N // _R,),
        in_specs=[
            pl.BlockSpec((_R, D), lambda i: (i, 0)),
            pl.BlockSpec((D, D), lambda i: (0, 0)),
            pl.BlockSpec((NC, ACC_N), lambda i: (0, 0)),
        ],
        out_specs=pl.BlockSpec((_R, D), lambda i: (i, 0)),
        out_shape=jax.ShapeDtypeStruct((ACC_N, D), jnp.float32),
    )(x, W, degp)


@functools.partial(
    pl.kernel,
    out_type=jax.ShapeDtypeStruct((NC, ACC_N, D), jnp.float32),
    mesh=_mesh,
    scratch_types=[
        pltpu.VMEM((B, D), jnp.float32),
        pltpu.VMEM((B, D), jnp.float32),
        pltpu.VMEM((NBT, B), jnp.int32),
        pltpu.VMEM((2, B), jnp.int32),
        pltpu.VMEM_SHARED((ACC_N, D), jnp.float32),
        pltpu.SemaphoreType.DMA,
        pltpu.SemaphoreType.DMA,
        pltpu.SemaphoreType.DMA,
        pltpu.SemaphoreType.DMA,
    ],
)
def _agg_call(hs_hbm, src_hbm, dst_hbm, z_hbm, out_hbm,
              buf0, buf1, srcv, dstr, acc, g0, g1, d0, d1):
    c = lax.axis_index("c")
    s = lax.axis_index("s")
    wid = c * NS + s
    row0 = wid * NBT
    pltpu.sync_copy(src_hbm.at[pl.ds(row0, NBT)], srcv)
    pltpu.sync_copy(z_hbm, acc.at[pl.ds(s * ROWS_PT, ROWS_PT)])
    plsc.subcore_barrier()

    dsems = (d0, d1)
    gsems = (g0, g1)
    bufs = (buf0, buf1)

    def _dfetch(j, slot):
        pltpu.async_copy(dst_hbm.at[row0 + j, 0], dstr.at[slot], dsems[slot])

    def _dwait(j, slot):
        pltpu.make_async_copy(dst_hbm.at[row0 + j, 0], dstr.at[slot],
                              dsems[slot]).wait()

    def _gstart(j, slot):
        pltpu.async_copy(hs_hbm.at[srcv.at[j]], bufs[slot], gsems[slot])

    def _gwait(j, slot):
        pltpu.make_async_copy(hs_hbm.at[srcv.at[j]], bufs[slot],
                              gsems[slot]).wait()

    _dfetch(0, 0)
    _dfetch(1, 1)
    _gstart(0, 0)
    _gstart(1, 1)

    def body(jj, carry):
        j0 = jj * 2
        for slot in (0, 1):
            j = j0 + slot
            _gwait(j, slot)
            _dwait(j, slot)
            pltpu.sync_copy(bufs[slot], acc.at[dstr.at[slot]], add=True)

            @pl.when(j + 2 < NBT)
            def _():
                _dfetch(j + 2, slot)
                _gstart(j + 2, slot)

        return carry

    lax.fori_loop(0, NBT // 2, body, 0)
    plsc.subcore_barrier()
    pltpu.sync_copy(acc.at[pl.ds(s * ROWS_PT, ROWS_PT)],
                    out_hbm.at[c, pl.ds(s * ROWS_PT, ROWS_PT)])


def _final_body(acc_ref, hs_ref, degp_ref, b_ref, out_ref):
    i = pl.program_id(0)
    deg = (degp_ref[0, pl.ds(i * _R, _R)] + degp_ref[1, pl.ds(i * _R, _R)]
           + 1.0)
    dis = lax.rsqrt(deg)
    a = acc_ref[0] + acc_ref[1] + hs_ref[...]
    o = a * dis[:, None] + b_ref[...]
    m = jnp.max(o, axis=1, keepdims=True)
    lse = jnp.log(jnp.sum(jnp.exp(o - m), axis=1, keepdims=True)) + m
    out_ref[...] = o - lse


def _final_call(acc, hs, degp, b2):
    return pl.pallas_call(
        _final_body,
        grid=(ACC_N // _R,),
        in_specs=[
            pl.BlockSpec((NC, _R, D), lambda i: (0, i, 0)),
            pl.BlockSpec((_R, D), lambda i: (i, 0)),
            pl.BlockSpec((NC, ACC_N), lambda i: (0, 0)),
            pl.BlockSpec((1, D), lambda i: (0, 0)),
        ],
        out_specs=pl.BlockSpec((_R, D), lambda i: (i, 0)),
        out_shape=jax.ShapeDtypeStruct((ACC_N, D), jnp.float32),
    )(acc, hs, degp, b2)


def kernel(x, edge_index, W, b):
    x_p = jnp.concatenate(
        [x, jnp.zeros((ACC_N - N, D), jnp.float32)], axis=0)
    src2 = edge_index[0].reshape(E // B, B)
    dst2 = edge_index[1].reshape(E // B, B)
    dst3 = edge_index[1].reshape(E // B, 1, B)
    degp = _deg_call(dst2,
                     jnp.ones((B,), jnp.float32),
                     jnp.zeros((ROWS_PT,), jnp.float32))
    hs = _hs_call(x_p, W, degp)
    acc = _agg_call(hs, src2, dst3,
                    jnp.zeros((ROWS_PT, D), jnp.float32))
    return _final_call(acc, hs, degp, b.reshape(1, D))[:N]
